# baseline (speedup 1.0000x reference)
import jax
import jax.numpy as jnp
from jax import lax
from jax.experimental import pallas as pl
from jax.experimental.pallas import tpu as pltpu
from jax.experimental.pallas import tpu_sc as plsc

_B, _C, _H, _W = 128, 10, 128, 128
_HW = _H * _W
_NC, _NS, _L = 2, 16, 16
_NWORK = _NC * _NS
_BPW = _B // _NWORK
_SLAB = 2
_NSLAB = _C // _SLAB

def _mesh():
    return plsc.VectorSubcoreMesh(core_axis_name="c", subcore_axis_name="s")


def _lanes():
    return lax.broadcasted_iota(jnp.int32, (_L,), 0)


_GDN = lax.GatherDimensionNumbers(
    offset_dims=(), collapsed_slice_dims=(0,), start_index_map=(0,))


def _shuffle(v, idx):
    return lax.gather(v, idx[:, None], _GDN, (1,),
                      mode=lax.GatherScatterMode.PROMISE_IN_BOUNDS)


def _splat_sum(v):
    idx = _lanes()
    for sh in (1, 2, 4, 8):
        v = v + _shuffle(v, jnp.bitwise_xor(idx, sh))
    return v


def _row_sum(buf, r):
    z = jnp.zeros((_L,), jnp.float32)

    def body(i, accs):
        a0, a1, a2, a3 = accs
        base = i * 64
        a0 = a0 + buf[r, pl.ds(base, _L)]
        a1 = a1 + buf[r, pl.ds(base + 16, _L)]
        a2 = a2 + buf[r, pl.ds(base + 32, _L)]
        a3 = a3 + buf[r, pl.ds(base + 48, _L)]
        return a0, a1, a2, a3

    a0, a1, a2, a3 = lax.fori_loop(0, _HW // 64, body, (z, z, z, z))
    return _splat_sum((a0 + a1) + (a2 + a3))


def _fixup_rows(fbg, ftgt):
    def body(i, carry):
        base = i * _L
        g = fbg[0, pl.ds(base, _L)]
        t = ftgt[0, pl.ds(base, _L)]
        m = g > 0.5
        fbg[0, pl.ds(base, _L)] = jnp.where(m, 0.0, g)
        ftgt[0, pl.ds(base, _L)] = jnp.where(m, 1.0, t)
        return carry

    lax.fori_loop(0, _HW // _L, body, 0, unroll=8)


def _sc_body(g_ref, t_ref, out_ref,
             buf_a, buf_b, fbg, ftgt, tbuf, idxb, idxt,
             rs_a, rs_b, ws_a, ws_b):
    cid = lax.axis_index("c")
    sid = lax.axis_index("s")
    wid = sid * _NC + cid
    pltpu.sync_copy(t_ref, tbuf)
    tgt_vec = tbuf[...]
    lane0 = _lanes() == 0

    bufs = (buf_a, buf_b)
    rsem = (rs_a, rs_b)
    wsem = (ws_a, ws_b)

    for j in range(_BPW):
        b = wid * _BPW + j
        row0 = b * _C
        rcp = [None] * _NSLAB
        wcp = [None] * _NSLAB
        rcp[0] = pltpu.async_copy(
            g_ref.at[pl.ds(row0, _SLAB)], bufs[0], rsem[0])
        best = jnp.zeros((_L,), jnp.float32)
        bgc = jnp.zeros((_L,), jnp.int32)
        for s in range(_NSLAB):
            pbuf = bufs[s % 2]
            if s + 1 < _NSLAB:
                nxt = (s + 1) % 2
                if s >= 1:
                    wcp[s - 1].wait()
                rcp[s + 1] = pltpu.async_copy(
                    g_ref.at[pl.ds(row0 + (s + 1) * _SLAB, _SLAB)],
                    bufs[nxt], rsem[nxt])
            rcp[s].wait()
            wcp[s] = pltpu.async_copy(
                pbuf, out_ref.at[pl.ds(row0 + s * _SLAB, _SLAB)],
                wsem[s % 2])
            for r in range(_SLAB):
                c = s * _SLAB + r
                sc = _row_sum(pbuf, r)
                if c == 0:
                    best = sc
                else:
                    upd = sc > best
                    best = jnp.where(upd, sc, best)
                    bgc = jnp.where(upd, jnp.int32(c), bgc)
        wcp[_NSLAB - 2].wait()
        wcp[_NSLAB - 1].wait()
        zidx = jnp.zeros((_L,), jnp.int32)
        plsc.store_scatter(idxb, [zidx], bgc + row0, mask=lane0)
        plsc.store_scatter(idxt, [zidx], tgt_vec + row0, mask=lane0)
        pltpu.async_copy(g_ref.at[idxb], fbg, rs_a).wait()
        pltpu.async_copy(g_ref.at[idxt], ftgt, rs_b).wait()
        _fixup_rows(fbg, ftgt)
        pltpu.async_copy(fbg, out_ref.at[idxb], ws_a).wait()
        pltpu.async_copy(ftgt, out_ref.at[idxt], ws_b).wait()


def kernel(grid, target_color):
    g2 = grid.reshape(_B * _C, _HW)
    tgt16 = jnp.full((_L,), target_color, jnp.int32)
    sck = pl.kernel(
        _sc_body,
        out_type=jax.ShapeDtypeStruct((_B * _C, _HW), jnp.float32),
        mesh=_mesh(),
        compiler_params=pltpu.CompilerParams(needs_layout_passes=False),
        scratch_types=[
            pltpu.VMEM((_SLAB, _HW), jnp.float32),
            pltpu.VMEM((_SLAB, _HW), jnp.float32),
            pltpu.VMEM((1, _HW), jnp.float32),
            pltpu.VMEM((1, _HW), jnp.float32),
            pltpu.VMEM((_L,), jnp.int32),
            pltpu.VMEM((1,), jnp.int32),
            pltpu.VMEM((1,), jnp.int32),
            pltpu.SemaphoreType.DMA,
            pltpu.SemaphoreType.DMA,
            pltpu.SemaphoreType.DMA,
            pltpu.SemaphoreType.DMA,
        ],
    )
    out = sck(g2, tgt16)
    return out.reshape(_B, _C, _H, _W)


# speedup vs baseline: 2.7227x; 2.7227x over previous
import jax
import jax.numpy as jnp
from jax import lax
from jax.experimental import pallas as pl
from jax.experimental.pallas import tpu as pltpu
from jax.experimental.pallas import tpu_sc as plsc

_B, _C, _H, _W = 128, 10, 128, 128
_NC, _NS, _L = 2, 16, 16
_NWORK = _NC * _NS
_BPW = _B // _NWORK
_SLAB = 2
_NSLAB = _C // _SLAB


def _mesh():
    return plsc.VectorSubcoreMesh(core_axis_name="c", subcore_axis_name="s")


def _row_sum(buf, r):
    z = jnp.zeros((_L,), jnp.float32)

    def body(h, accs):
        a0, a1, a2, a3 = accs
        a0 = a0 + buf[r, h, pl.ds(0, _L)] + buf[r, h, pl.ds(64, _L)]
        a1 = a1 + buf[r, h, pl.ds(16, _L)] + buf[r, h, pl.ds(80, _L)]
        a2 = a2 + buf[r, h, pl.ds(32, _L)] + buf[r, h, pl.ds(96, _L)]
        a3 = a3 + buf[r, h, pl.ds(48, _L)] + buf[r, h, pl.ds(112, _L)]
        return a0, a1, a2, a3

    a0, a1, a2, a3 = lax.fori_loop(0, _H, body, (z, z, z, z))
    return jnp.sum((a0 + a1) + (a2 + a3))


def _fixup_planes(fbg, ftgt):
    def body(h, carry):
        for k in range(_W // _L):
            g = fbg[h, pl.ds(k * _L, _L)]
            t = ftgt[h, pl.ds(k * _L, _L)]
            m = g > 0.5
            fbg[h, pl.ds(k * _L, _L)] = jnp.where(m, 0.0, g)
            ftgt[h, pl.ds(k * _L, _L)] = jnp.where(m, 1.0, t)
        return carry

    lax.fori_loop(0, _H, body, 0)


def _sc_body(g_ref, t_ref, out_ref,
             buf_a, buf_b, fbg, ftgt, tbuf,
             rs_a, rs_b, ws_a, ws_b):
    cid = lax.axis_index("c")
    sid = lax.axis_index("s")
    wid = sid * _NC + cid
    pltpu.sync_copy(t_ref, tbuf)
    tgt = jnp.max(tbuf[...]).astype(jnp.int32)

    bufs = (buf_a, buf_b)
    rsem = (rs_a, rs_b)
    wsem = (ws_a, ws_b)

    for j in range(_BPW):
        b = wid * _BPW + j
        rcp = [None] * _NSLAB
        wcp = [None] * _NSLAB
        rcp[0] = pltpu.async_copy(
            g_ref.at[b, pl.ds(0, _SLAB)], bufs[0], rsem[0])
        best = jnp.float32(0.0)
        bgc = jnp.int32(0)
        for s in range(_NSLAB):
            pbuf = bufs[s % 2]
            if s + 1 < _NSLAB:
                nxt = (s + 1) % 2
                if s >= 1:
                    wcp[s - 1].wait()
                rcp[s + 1] = pltpu.async_copy(
                    g_ref.at[b, pl.ds((s + 1) * _SLAB, _SLAB)],
                    bufs[nxt], rsem[nxt])
            rcp[s].wait()
            wcp[s] = pltpu.async_copy(
                pbuf, out_ref.at[b, pl.ds(s * _SLAB, _SLAB)], wsem[s % 2])
            for r in range(_SLAB):
                c = s * _SLAB + r
                sc = _row_sum(pbuf, r)
                if c == 0:
                    best = sc
                else:
                    upd = sc > best
                    best = jnp.where(upd, sc, best)
                    bgc = jnp.where(upd, jnp.int32(c), bgc)
        wcp[_NSLAB - 2].wait()
        wcp[_NSLAB - 1].wait()
        pltpu.sync_copy(g_ref.at[b, bgc], fbg)
        pltpu.sync_copy(g_ref.at[b, tgt], ftgt)
        _fixup_planes(fbg, ftgt)
        pltpu.sync_copy(fbg, out_ref.at[b, bgc])
        pltpu.sync_copy(ftgt, out_ref.at[b, tgt])


def kernel(grid, target_color):
    tgt16 = jnp.full((_L,), target_color, jnp.float32)
    sck = pl.kernel(
        _sc_body,
        out_type=jax.ShapeDtypeStruct((_B, _C, _H, _W), jnp.float32),
        mesh=_mesh(),
        compiler_params=pltpu.CompilerParams(needs_layout_passes=False),
        scratch_types=[
            pltpu.VMEM((_SLAB, _H, _W), jnp.float32),
            pltpu.VMEM((_SLAB, _H, _W), jnp.float32),
            pltpu.VMEM((_H, _W), jnp.float32),
            pltpu.VMEM((_H, _W), jnp.float32),
            pltpu.VMEM((_L,), jnp.float32),
            pltpu.SemaphoreType.DMA,
            pltpu.SemaphoreType.DMA,
            pltpu.SemaphoreType.DMA,
            pltpu.SemaphoreType.DMA,
        ],
    )
    return sck(grid, tgt16)


# speedup vs baseline: 2.8477x; 1.0459x over previous
import jax
import jax.numpy as jnp
from jax import lax
from jax.experimental import pallas as pl
from jax.experimental.pallas import tpu as pltpu
from jax.experimental.pallas import tpu_sc as plsc

_B, _C, _H, _W = 128, 10, 128, 128
_NC, _NS, _L = 2, 16, 16
_NWORK = _NC * _NS
_BPW = _B // _NWORK
_SLAB = 2
_NSLAB = _C // _SLAB


def _mesh():
    return plsc.VectorSubcoreMesh(core_axis_name="c", subcore_axis_name="s")


def _row_sum(buf, r):
    z = jnp.zeros((_L,), jnp.float32)

    def body(hh, accs):
        a0, a1, a2, a3 = accs
        h = hh * 2
        for hi in (0, 1):
            a0 = a0 + buf[r, h + hi, pl.ds(0, _L)] + buf[r, h + hi, pl.ds(64, _L)]
            a1 = a1 + buf[r, h + hi, pl.ds(16, _L)] + buf[r, h + hi, pl.ds(80, _L)]
            a2 = a2 + buf[r, h + hi, pl.ds(32, _L)] + buf[r, h + hi, pl.ds(96, _L)]
            a3 = a3 + buf[r, h + hi, pl.ds(48, _L)] + buf[r, h + hi, pl.ds(112, _L)]
        return a0, a1, a2, a3

    a0, a1, a2, a3 = lax.fori_loop(0, _H // 2, body, (z, z, z, z))
    return jnp.sum((a0 + a1) + (a2 + a3))


def _fixup_planes(fbg, ftgt, bgval):
    def body(h, carry):
        for k in range(_W // _L):
            g = fbg[h, pl.ds(k * _L, _L)]
            t = ftgt[h, pl.ds(k * _L, _L)]
            m = g > 0.5
            fbg[h, pl.ds(k * _L, _L)] = jnp.where(m, bgval, g)
            ftgt[h, pl.ds(k * _L, _L)] = jnp.where(m, 1.0, t)
        return carry

    lax.fori_loop(0, _H, body, 0)


def _sc_body(g_ref, t_ref, out_ref,
             buf_a, buf_b, fbg, ftgt, tbuf,
             rs_a, rs_b, ws_a, ws_b, fs_a, fs_b, fw_a, fw_b):
    cid = lax.axis_index("c")
    sid = lax.axis_index("s")
    wid = sid * _NC + cid
    pltpu.sync_copy(t_ref, tbuf)
    tgt = jnp.max(tbuf[...]).astype(jnp.int32)

    bufs = (buf_a, buf_b)
    rsem = (rs_a, rs_b)
    wsem = (ws_a, ws_b)

    fix_pending = None
    fix_writes = None

    def run_fixup(pending, writes):
        b_p, bgc_p, gb, gt = pending
        if writes is not None:
            writes[0].wait()
            writes[1].wait()
        gb.wait()
        gt.wait()
        bgval = jnp.where(bgc_p == tgt, 1.0, 0.0)
        _fixup_planes(fbg, ftgt, bgval)
        wb = pltpu.async_copy(fbg, out_ref.at[b_p, bgc_p], fw_a)
        wt = pltpu.async_copy(ftgt, out_ref.at[b_p, tgt], fw_b)
        return (wb, wt)

    for j in range(_BPW):
        b = wid * _BPW + j
        rcp = [None] * _NSLAB
        wcp = [None] * _NSLAB
        rcp[0] = pltpu.async_copy(
            g_ref.at[b, pl.ds(0, _SLAB)], bufs[0], rsem[0])
        if fix_pending is not None:
            fix_writes = run_fixup(fix_pending, fix_writes)
            fix_pending = None
        best = jnp.float32(0.0)
        bgc = jnp.int32(0)
        for s in range(_NSLAB):
            pbuf = bufs[s % 2]
            if s + 1 < _NSLAB:
                nxt = (s + 1) % 2
                if s >= 1:
                    wcp[s - 1].wait()
                rcp[s + 1] = pltpu.async_copy(
                    g_ref.at[b, pl.ds((s + 1) * _SLAB, _SLAB)],
                    bufs[nxt], rsem[nxt])
            rcp[s].wait()
            wcp[s] = pltpu.async_copy(
                pbuf, out_ref.at[b, pl.ds(s * _SLAB, _SLAB)], wsem[s % 2])
            for r in range(_SLAB):
                c = s * _SLAB + r
                sc = _row_sum(pbuf, r)
                if c == 0:
                    best = sc
                else:
                    upd = sc > best
                    best = jnp.where(upd, sc, best)
                    bgc = jnp.where(upd, jnp.int32(c), bgc)
        wcp[_NSLAB - 2].wait()
        wcp[_NSLAB - 1].wait()
        gb = pltpu.async_copy(g_ref.at[b, bgc], fbg, fs_a)
        gt = pltpu.async_copy(g_ref.at[b, tgt], ftgt, fs_b)
        fix_pending = (b, bgc, gb, gt)

    fix_writes = run_fixup(fix_pending, fix_writes)
    fix_writes[0].wait()
    fix_writes[1].wait()


def kernel(grid, target_color):
    tgt16 = jnp.full((_L,), target_color, jnp.float32)
    sck = pl.kernel(
        _sc_body,
        out_type=jax.ShapeDtypeStruct((_B, _C, _H, _W), jnp.float32),
        mesh=_mesh(),
        compiler_params=pltpu.CompilerParams(needs_layout_passes=False),
        scratch_types=[
            pltpu.VMEM((_SLAB, _H, _W), jnp.float32),
            pltpu.VMEM((_SLAB, _H, _W), jnp.float32),
            pltpu.VMEM((_H, _W), jnp.float32),
            pltpu.VMEM((_H, _W), jnp.float32),
            pltpu.VMEM((_L,), jnp.float32),
            pltpu.SemaphoreType.DMA,
            pltpu.SemaphoreType.DMA,
            pltpu.SemaphoreType.DMA,
            pltpu.SemaphoreType.DMA,
            pltpu.SemaphoreType.DMA,
            pltpu.SemaphoreType.DMA,
            pltpu.SemaphoreType.DMA,
            pltpu.SemaphoreType.DMA,
        ],
    )
    return sck(grid, tgt16)
